# Initial kernel scaffold; baseline (speedup 1.0000x reference)
import jax
import jax.numpy as jnp
from jax.experimental import pallas as pl


def kernel(output, target):
    raise NotImplementedError("write your pallas kernel here")


# speedup vs baseline: 4.7523x; 4.7523x over previous
import functools

import jax
import jax.numpy as jnp
from jax import lax
from jax.experimental import pallas as pl
from jax.experimental.pallas import tpu as pltpu

C = 19
N = 8
H = 512
W = 512
HB = 128
EPS = 1e-4


def _dice_body(out_ref, tgt_ref, loss_ref, acc_ref):
    n = pl.program_id(0)
    h = pl.program_id(1)

    @pl.when(jnp.logical_and(n == 0, h == 0))
    def _init():
        acc_ref[...] = jnp.zeros_like(acc_ref)

    e = jnp.exp(out_ref[0])
    t = tgt_ref[0]
    cidx = lax.broadcasted_iota(jnp.int32, (C, HB, W), 0)
    mask = t[None, :, :] == cidx
    inter = jnp.sum(jnp.where(mask, e, 0.0), axis=1)
    denom = jnp.sum(e, axis=1)
    hist = jnp.sum(jnp.where(mask, 1.0, 0.0), axis=1)
    acc_ref[0:C, :] += inter
    acc_ref[C:2 * C, :] += denom
    acc_ref[2 * C:3 * C, :] += hist

    @pl.when(jnp.logical_and(n == pl.num_programs(0) - 1,
                             h == pl.num_programs(1) - 1))
    def _fin():
        inter_t = jnp.sum(acc_ref[0:C, :], axis=1)
        sum_e = jnp.sum(acc_ref[C:2 * C, :], axis=1)
        hist_t = jnp.sum(acc_ref[2 * C:3 * C, :], axis=1)
        total = float(N * H * W)
        weights = 1.0 / jnp.log(1.1 + hist_t / total)
        numerator = 2.0 * inter_t
        denominator = sum_e + hist_t + EPS
        loss = jnp.sum(weights * (1.0 - numerator / denominator)) / float(N)
        loss_ref[...] = jnp.broadcast_to(loss, (1, 1))


@functools.partial(jax.jit, static_argnames=())
def kernel(output, target):
    target = target.astype(jnp.int32)
    grid = (N, H // HB)
    res = pl.pallas_call(
        _dice_body,
        grid=grid,
        in_specs=[
            pl.BlockSpec((1, C, HB, W), lambda n, h: (n, 0, h, 0)),
            pl.BlockSpec((1, HB, W), lambda n, h: (n, h, 0)),
        ],
        out_specs=pl.BlockSpec((1, 1), lambda n, h: (0, 0)),
        out_shape=jax.ShapeDtypeStruct((1, 1), jnp.float32),
        scratch_shapes=[pltpu.VMEM((3 * C, W), jnp.float32)],
        compiler_params=pltpu.CompilerParams(
            dimension_semantics=("arbitrary", "arbitrary"),
        ),
    )(output, target)
    return res[0, 0]


# speedup vs baseline: 5.5238x; 1.1623x over previous
import functools

import jax
import jax.numpy as jnp
from jax import lax
from jax.experimental import pallas as pl
from jax.experimental.pallas import tpu as pltpu

C = 19
N = 8
H = 512
W = 512
HB = 128
EPS = 1e-4

_DN = (((0,), (0,)), ((), ()))


def _dice_body(out_ref, tgt_ref, loss_ref, acc_ref):
    n = pl.program_id(0)
    h = pl.program_id(1)

    @pl.when(jnp.logical_and(n == 0, h == 0))
    def _init():
        acc_ref[...] = jnp.zeros_like(acc_ref)

    e = jnp.exp(out_ref[0])
    t = tgt_ref[0]
    ones = jnp.ones((HB, 1), jnp.float32)
    for c in range(C):
        ec = e[c]
        mc = t == c
        sel = jnp.where(mc, ec, 0.0)
        hf = jnp.where(mc, 1.0, 0.0)
        inter = lax.dot_general(ones, sel, _DN, preferred_element_type=jnp.float32)
        denom = lax.dot_general(ones, ec, _DN, preferred_element_type=jnp.float32)
        hist = lax.dot_general(ones, hf, _DN, preferred_element_type=jnp.float32)
        acc_ref[c:c + 1, :] += inter
        acc_ref[C + c:C + c + 1, :] += denom
        acc_ref[2 * C + c:2 * C + c + 1, :] += hist

    @pl.when(jnp.logical_and(n == pl.num_programs(0) - 1,
                             h == pl.num_programs(1) - 1))
    def _fin():
        inter_t = jnp.sum(acc_ref[0:C, :], axis=1)
        sum_e = jnp.sum(acc_ref[C:2 * C, :], axis=1)
        hist_t = jnp.sum(acc_ref[2 * C:3 * C, :], axis=1)
        total = float(N * H * W)
        weights = 1.0 / jnp.log(1.1 + hist_t / total)
        numerator = 2.0 * inter_t
        denominator = sum_e + hist_t + EPS
        loss = jnp.sum(weights * (1.0 - numerator / denominator)) / float(N)
        loss_ref[...] = jnp.broadcast_to(loss, (1, 1))


@functools.partial(jax.jit, static_argnames=())
def kernel(output, target):
    target = target.astype(jnp.int32)
    grid = (N, H // HB)
    res = pl.pallas_call(
        _dice_body,
        grid=grid,
        in_specs=[
            pl.BlockSpec((1, C, HB, W), lambda n, h: (n, 0, h, 0)),
            pl.BlockSpec((1, HB, W), lambda n, h: (n, h, 0)),
        ],
        out_specs=pl.BlockSpec((1, 1), lambda n, h: (0, 0)),
        out_shape=jax.ShapeDtypeStruct((1, 1), jnp.float32),
        scratch_shapes=[pltpu.VMEM((3 * C, W), jnp.float32)],
        compiler_params=pltpu.CompilerParams(
            dimension_semantics=("arbitrary", "arbitrary"),
        ),
    )(output, target)
    return res[0, 0]


# speedup vs baseline: 6.1054x; 1.1053x over previous
import functools

import jax
import jax.numpy as jnp
from jax import lax
from jax.experimental import pallas as pl
from jax.experimental.pallas import tpu as pltpu

C = 19
N = 8
H = 512
W = 512
HB = 256
EPS = 1e-4

_DN = (((0,), (0,)), ((), ()))


def _dice_body(out_ref, tgt_ref, loss_ref, acc_ref):
    n = pl.program_id(0)
    h = pl.program_id(1)

    @pl.when(jnp.logical_and(n == 0, h == 0))
    def _init():
        acc_ref[...] = jnp.zeros_like(acc_ref)

    e = jnp.exp(out_ref[0])
    t = tgt_ref[0]
    ones = jnp.ones((HB, 1), jnp.float32)
    for c in range(C):
        ec = e[c]
        mc = t == c
        sel = jnp.where(mc, ec, 0.0)
        hf = jnp.where(mc, 1.0, 0.0)
        inter = lax.dot_general(ones, sel, _DN, preferred_element_type=jnp.float32)
        denom = lax.dot_general(ones, ec, _DN, preferred_element_type=jnp.float32)
        hist = lax.dot_general(ones, hf, _DN, preferred_element_type=jnp.float32)
        acc_ref[c:c + 1, :] += inter
        acc_ref[C + c:C + c + 1, :] += denom
        acc_ref[2 * C + c:2 * C + c + 1, :] += hist

    @pl.when(jnp.logical_and(n == pl.num_programs(0) - 1,
                             h == pl.num_programs(1) - 1))
    def _fin():
        inter_t = jnp.sum(acc_ref[0:C, :], axis=1)
        sum_e = jnp.sum(acc_ref[C:2 * C, :], axis=1)
        hist_t = jnp.sum(acc_ref[2 * C:3 * C, :], axis=1)
        total = float(N * H * W)
        weights = 1.0 / jnp.log(1.1 + hist_t / total)
        numerator = 2.0 * inter_t
        denominator = sum_e + hist_t + EPS
        loss = jnp.sum(weights * (1.0 - numerator / denominator)) / float(N)
        loss_ref[...] = jnp.broadcast_to(loss, (1, 1))


@functools.partial(jax.jit, static_argnames=())
def kernel(output, target):
    target = target.astype(jnp.int32)
    grid = (N, H // HB)
    res = pl.pallas_call(
        _dice_body,
        grid=grid,
        in_specs=[
            pl.BlockSpec((1, C, HB, W), lambda n, h: (n, 0, h, 0)),
            pl.BlockSpec((1, HB, W), lambda n, h: (n, h, 0)),
        ],
        out_specs=pl.BlockSpec((1, 1), lambda n, h: (0, 0)),
        out_shape=jax.ShapeDtypeStruct((1, 1), jnp.float32),
        scratch_shapes=[pltpu.VMEM((3 * C, W), jnp.float32)],
        compiler_params=pltpu.CompilerParams(
            dimension_semantics=("arbitrary", "arbitrary"),
        ),
    )(output, target)
    return res[0, 0]


# speedup vs baseline: 6.1482x; 1.0070x over previous
import functools

import jax
import jax.numpy as jnp
from jax import lax
from jax.experimental import pallas as pl
from jax.experimental.pallas import tpu as pltpu

C = 19
N = 8
H = 512
W = 512
HB = 512
EPS = 1e-4

_DN = (((0,), (0,)), ((), ()))


def _dice_body(out_ref, tgt_ref, loss_ref, acc_ref):
    n = pl.program_id(0)
    h = pl.program_id(1)

    @pl.when(jnp.logical_and(n == 0, h == 0))
    def _init():
        acc_ref[...] = jnp.zeros_like(acc_ref)

    e = jnp.exp(out_ref[0])
    t = tgt_ref[0]
    ones = jnp.ones((HB, 1), jnp.float32)
    for c in range(C):
        ec = e[c]
        mc = t == c
        sel = jnp.where(mc, ec, 0.0)
        hf = jnp.where(mc, 1.0, 0.0)
        inter = lax.dot_general(ones, sel, _DN, preferred_element_type=jnp.float32)
        denom = lax.dot_general(ones, ec, _DN, preferred_element_type=jnp.float32)
        hist = lax.dot_general(ones, hf, _DN, preferred_element_type=jnp.float32)
        acc_ref[c:c + 1, :] += inter
        acc_ref[C + c:C + c + 1, :] += denom
        acc_ref[2 * C + c:2 * C + c + 1, :] += hist

    @pl.when(jnp.logical_and(n == pl.num_programs(0) - 1,
                             h == pl.num_programs(1) - 1))
    def _fin():
        inter_t = jnp.sum(acc_ref[0:C, :], axis=1)
        sum_e = jnp.sum(acc_ref[C:2 * C, :], axis=1)
        hist_t = jnp.sum(acc_ref[2 * C:3 * C, :], axis=1)
        total = float(N * H * W)
        weights = 1.0 / jnp.log(1.1 + hist_t / total)
        numerator = 2.0 * inter_t
        denominator = sum_e + hist_t + EPS
        loss = jnp.sum(weights * (1.0 - numerator / denominator)) / float(N)
        loss_ref[...] = jnp.broadcast_to(loss, (1, 1))


@functools.partial(jax.jit, static_argnames=())
def kernel(output, target):
    target = target.astype(jnp.int32)
    grid = (N, H // HB)
    res = pl.pallas_call(
        _dice_body,
        grid=grid,
        in_specs=[
            pl.BlockSpec((1, C, HB, W), lambda n, h: (n, 0, h, 0)),
            pl.BlockSpec((1, HB, W), lambda n, h: (n, h, 0)),
        ],
        out_specs=pl.BlockSpec((1, 1), lambda n, h: (0, 0)),
        out_shape=jax.ShapeDtypeStruct((1, 1), jnp.float32),
        scratch_shapes=[pltpu.VMEM((3 * C, W), jnp.float32)],
        compiler_params=pltpu.CompilerParams(
            dimension_semantics=("arbitrary", "arbitrary"),
        ),
    )(output, target)
    return res[0, 0]
